# baseline (speedup 1.0000x reference)
import functools

import jax
import jax.numpy as jnp
from jax import lax
from jax.experimental import pallas as pl
from jax.experimental.pallas import tpu as pltpu
from jax.experimental.pallas import tpu_sc as plsc

B = 16384
L = 50
VOCAB = 256
DIM = 256

_NC, _NS = 2, 16
_NW = _NC * _NS
_RPW = B // _NW
_SUB = 128
_NSUB = _RPW // _SUB
_GRP = _SUB // 16


def _sc_body(idx_hbm, psw_hbm, coeff_hbm, idx_v, psw_v, acc_v):
    wid = lax.axis_index("s") * _NC + lax.axis_index("c")
    base = wid * _RPW
    lane = lax.iota(jnp.int32, 16)
    zeros16 = jnp.zeros((16,), jnp.float32)

    for sub in range(_NSUB):
        r0 = base + sub * _SUB
        pltpu.sync_copy(idx_hbm.at[pl.ds(r0, _SUB)], idx_v)
        pltpu.sync_copy(psw_hbm.at[pl.ds(r0, _SUB)], psw_v)

        @plsc.parallel_loop(0, _SUB)
        def _zero(r):
            for k in range(VOCAB // 16):
                acc_v[r, pl.ds(k * 16, 16)] = zeros16

        @plsc.parallel_loop(0, L, unroll=2)
        def _scatter(l):
            cols = jnp.full((16,), l, jnp.int32)
            for g in range(_GRP):
                rows = lane + g * 16
                ivals = plsc.load_gather(idx_v, [rows, cols])
                pvals = plsc.load_gather(psw_v, [rows, cols])
                plsc.addupdate_scatter(acc_v, [rows, ivals], pvals)

        pltpu.sync_copy(acc_v, coeff_hbm.at[pl.ds(r0, _SUB)])


_sc_coeff = functools.partial(
    pl.kernel,
    out_type=jax.ShapeDtypeStruct((B, VOCAB), jnp.float32),
    mesh=plsc.VectorSubcoreMesh(
        core_axis_name="c", subcore_axis_name="s", num_cores=_NC, num_subcores=_NS
    ),
    scratch_types=[
        pltpu.VMEM((_SUB, L), jnp.int32),
        pltpu.VMEM((_SUB, L), jnp.float32),
        pltpu.VMEM((_SUB, VOCAB), jnp.float32),
    ],
    compiler_params=pltpu.CompilerParams(needs_layout_passes=False),
)(_sc_body)


_MBLK = 2048


def _mm_body(c_ref, w_ref, o_ref):
    o_ref[...] = jnp.dot(c_ref[...], w_ref[...], preferred_element_type=jnp.float32)


def _tc_matmul(coeff, weight):
    return pl.pallas_call(
        _mm_body,
        grid=(B // _MBLK,),
        in_specs=[
            pl.BlockSpec((_MBLK, VOCAB), lambda i: (i, 0)),
            pl.BlockSpec((VOCAB, DIM), lambda i: (0, 0)),
        ],
        out_specs=pl.BlockSpec((_MBLK, DIM), lambda i: (i, 0)),
        out_shape=jax.ShapeDtypeStruct((B, DIM), jnp.float32),
    )(coeff, weight)


def kernel(indices, per_sample_weights, weight):
    coeff = _sc_coeff(indices.astype(jnp.int32), per_sample_weights)
    return _tc_matmul(coeff, weight)


# speedup vs baseline: 1.6347x; 1.6347x over previous
import functools

import jax
import jax.numpy as jnp
from jax import lax
from jax.experimental import pallas as pl
from jax.experimental.pallas import tpu as pltpu
from jax.experimental.pallas import tpu_sc as plsc

B = 16384
L = 50
VOCAB = 256
DIM = 256

_NC, _NS = 2, 16
_NW = _NC * _NS
_RPW = B // _NW
_SUB = 128
_NSUB = _RPW // _SUB
_GRP = _SUB // 16


def _sc_body(idx_hbm, psw_hbm, coeff_hbm, idx_v, psw_v, acc_v):
    wid = lax.axis_index("s") * _NC + lax.axis_index("c")
    base = wid * _RPW
    lane = lax.iota(jnp.int32, 16)
    zeros16 = jnp.zeros((16,), jnp.float32)

    pltpu.sync_copy(idx_hbm.at[:, pl.ds(base, _RPW)], idx_v)
    pltpu.sync_copy(psw_hbm.at[:, pl.ds(base, _RPW)], psw_v)

    for sub in range(_NSUB):
        r0 = base + sub * _SUB

        @plsc.parallel_loop(0, _SUB)
        def _zero(r):
            for k in range(VOCAB // 16):
                acc_v[r, pl.ds(k * 16, 16)] = zeros16

        @plsc.parallel_loop(0, L, unroll=2)
        def _scatter(l):
            for g in range(_GRP):
                s0 = sub * _SUB + g * 16
                ivals = idx_v[l, pl.ds(s0, 16)]
                pvals = psw_v[l, pl.ds(s0, 16)]
                rows = lane + g * 16
                plsc.addupdate_scatter(acc_v, [rows, ivals], pvals)

        pltpu.sync_copy(acc_v, coeff_hbm.at[pl.ds(r0, _SUB)])


_sc_coeff = functools.partial(
    pl.kernel,
    out_type=jax.ShapeDtypeStruct((B, VOCAB), jnp.float32),
    mesh=plsc.VectorSubcoreMesh(
        core_axis_name="c", subcore_axis_name="s", num_cores=_NC, num_subcores=_NS
    ),
    scratch_types=[
        pltpu.VMEM((L, _RPW), jnp.int32),
        pltpu.VMEM((L, _RPW), jnp.float32),
        pltpu.VMEM((_SUB, VOCAB), jnp.float32),
    ],
    compiler_params=pltpu.CompilerParams(needs_layout_passes=False),
)(_sc_body)


_MBLK = 2048


def _mm_body(c_ref, w_ref, o_ref):
    o_ref[...] = jnp.dot(c_ref[...], w_ref[...], preferred_element_type=jnp.float32)


def _tc_matmul(coeff, weight):
    return pl.pallas_call(
        _mm_body,
        grid=(B // _MBLK,),
        in_specs=[
            pl.BlockSpec((_MBLK, VOCAB), lambda i: (i, 0)),
            pl.BlockSpec((VOCAB, DIM), lambda i: (0, 0)),
        ],
        out_specs=pl.BlockSpec((_MBLK, DIM), lambda i: (i, 0)),
        out_shape=jax.ShapeDtypeStruct((B, DIM), jnp.float32),
    )(coeff, weight)


def kernel(indices, per_sample_weights, weight):
    coeff = _sc_coeff(indices.astype(jnp.int32).T, per_sample_weights.T)
    return _tc_matmul(coeff, weight)
